# baseline (speedup 1.0000x reference)
import functools

import jax
import jax.numpy as jnp
from jax import lax
from jax.experimental import pallas as pl
from jax.experimental.pallas import tpu as pltpu
from jax.experimental.pallas import tpu_sc as plsc

N = 10000
E = 160000
D = 256
H = 128
NC = 2
NS = 16
CH = 128
CPT = -(-E // (NS * CH))
EPT = CPT * CH
E2 = NS * EPT
RPT = (N // NS) // 8 * 8
REXTRA = N - NS * RPT


def _sc_agg_body(x_hbm, src_hbm, dst_hbm, out_hbm, srcv, dstb0, dstb1, rows0,
                 rows1, acc, sem0, sem1):
    c = lax.axis_index("c")
    s = lax.axis_index("s")
    roff = c * N

    pltpu.sync_copy(src_hbm.at[pl.ds(c * E2 + s * EPT, EPT)], srcv)

    r0 = s * RPT
    pltpu.sync_copy(x_hbm.at[pl.ds(roff + r0, RPT)], acc.at[pl.ds(r0, RPT)])

    @pl.when(s == NS - 1)
    def _init_extra():
        pltpu.sync_copy(x_hbm.at[pl.ds(roff + NS * RPT, REXTRA)],
                        acc.at[pl.ds(NS * RPT, REXTRA)])

    plsc.subcore_barrier()

    dbase = s * EPT

    def g_start(dstb, rowsb, semb, j):
        pltpu.async_copy(x_hbm.at[srcv.at[pl.ds(j * CH, CH)]], rowsb, semb)
        pltpu.sync_copy(dst_hbm.at[pl.ds(dbase + j * CH, CH)], dstb)

    def g_wait(rowsb, semb, j):
        pltpu.make_async_copy(x_hbm.at[srcv.at[pl.ds(j * CH, CH)]], rowsb,
                              semb).wait()

    def scat(dstb, rowsb):
        pltpu.sync_copy(rowsb, acc.at[dstb], add=True)

    g_start(dstb0, rows0, sem0, 0)

    @pl.loop(0, CPT // 2)
    def _chunks(jj):
        j = jj * 2
        g_wait(rows0, sem0, j)
        g_start(dstb1, rows1, sem1, j + 1)
        scat(dstb0, rows0)
        g_start(dstb0, rows0, sem0, j + 2)
        g_wait(rows1, sem1, j + 1)
        scat(dstb1, rows1)

    g_wait(rows0, sem0, CPT - 1)
    scat(dstb0, rows0)

    plsc.subcore_barrier()
    pltpu.sync_copy(acc.at[pl.ds(r0, RPT)], out_hbm.at[pl.ds(roff + r0, RPT)])

    @pl.when(s == NS - 1)
    def _out_extra():
        pltpu.sync_copy(acc.at[pl.ds(NS * RPT, REXTRA)],
                        out_hbm.at[pl.ds(roff + NS * RPT, REXTRA)])


@jax.jit
def _sc_agg(x_cat, src_cat, dst_pad):
    mesh = plsc.VectorSubcoreMesh(core_axis_name="c", subcore_axis_name="s")
    return pl.kernel(
        _sc_agg_body,
        out_type=jax.ShapeDtypeStruct((2 * N, H), jnp.float32),
        mesh=mesh,
        scratch_types=[
            pltpu.VMEM((EPT,), jnp.int32),
            pltpu.VMEM((CH,), jnp.int32),
            pltpu.VMEM((CH,), jnp.int32),
            pltpu.VMEM((CH, H), jnp.float32),
            pltpu.VMEM((CH, H), jnp.float32),
            pltpu.VMEM_SHARED((N + 8, H), jnp.float32),
            pltpu.SemaphoreType.DMA,
            pltpu.SemaphoreType.DMA,
        ],
    )(x_cat, src_cat, dst_pad)


def _dense_body_split(hin_ref, w_ref, b_ref, out_ref):
    hl = hin_ref[0]
    hh = hin_ref[1]
    h = (jnp.dot(hl, w_ref[:H, :], preferred_element_type=jnp.float32)
         + jnp.dot(hh, w_ref[H:, :], preferred_element_type=jnp.float32)
         + b_ref[...])
    o = h * jax.nn.sigmoid(h)
    out_ref[0] = o[:, :H]
    out_ref[1] = o[:, H:]


def _dense_body_last(hin_ref, w_ref, b_ref, out_ref):
    hl = hin_ref[0]
    hh = hin_ref[1]
    h = (jnp.dot(hl, w_ref[:H, :], preferred_element_type=jnp.float32)
         + jnp.dot(hh, w_ref[H:, :], preferred_element_type=jnp.float32)
         + b_ref[...])
    out_ref[...] = h * jax.nn.sigmoid(h)


_RB = 2000


@functools.partial(jax.jit, static_argnames=("last",))
def _dense(hin2, w, b2, last=False):
    grid = (N // _RB,)
    in_specs = [
        pl.BlockSpec((2, _RB, H), lambda i: (0, i, 0)),
        pl.BlockSpec((D, D), lambda i: (0, 0)),
        pl.BlockSpec((1, D), lambda i: (0, 0)),
    ]
    if last:
        return pl.pallas_call(
            _dense_body_last,
            grid=grid,
            in_specs=in_specs,
            out_specs=pl.BlockSpec((_RB, D), lambda i: (i, 0)),
            out_shape=jax.ShapeDtypeStruct((N, D), jnp.float32),
        )(hin2, w, b2)
    return pl.pallas_call(
        _dense_body_split,
        grid=grid,
        in_specs=in_specs,
        out_specs=pl.BlockSpec((2, _RB, H), lambda i: (0, i, 0)),
        out_shape=jax.ShapeDtypeStruct((2, N, H), jnp.float32),
    )(hin2, w, b2)


def kernel(x, edge_index, W0, b0, W1, b1, W2, b2):
    edges = edge_index.astype(jnp.int32)
    pad = jnp.zeros((E2 - E,), jnp.int32)
    src_pad = jnp.concatenate([edges[0], pad])
    src_cat = jnp.concatenate([src_pad, src_pad + N])
    dst_pad = jnp.concatenate([edges[1], pad + N])
    h2 = x.reshape(N, 2, H).transpose(1, 0, 2)
    params = [(W0, b0), (W1, b1), (W2, b2)]
    for li, (w, b) in enumerate(params):
        hin = _sc_agg(h2.reshape(2 * N, H), src_cat, dst_pad)
        h2 = _dense(hin.reshape(2, N, H), w, b.reshape(1, D), last=(li == 2))
    return h2


# speedup vs baseline: 1.2051x; 1.2051x over previous
import functools

import jax
import jax.numpy as jnp
from jax import lax
from jax.experimental import pallas as pl
from jax.experimental.pallas import tpu as pltpu
from jax.experimental.pallas import tpu_sc as plsc

N = 10000
E = 160000
D = 256
H = 128
NC = 2
NS = 16
EPT = E // NS
CH = 96
NFULL = EPT // CH
TAIL = EPT - NFULL * CH
RPT = (N // NS) // 8 * 8
REXTRA = N - NS * RPT


def _sc_agg_body(x_hbm, src_hbm, dst_hbm, out_hbm, src0, dst0, rows0, src1,
                 dst1, rows1, src2, dst2, rows2, tsrc, tdst, trows, acc, sem0,
                 sem1, sem2):
    c = lax.axis_index("c")
    s = lax.axis_index("s")
    roff = c * N

    r0 = s * RPT
    pltpu.sync_copy(x_hbm.at[pl.ds(roff + r0, RPT)], acc.at[pl.ds(r0, RPT)])

    @pl.when(s == NS - 1)
    def _init_extra():
        pltpu.sync_copy(x_hbm.at[pl.ds(roff + NS * RPT, REXTRA)],
                        acc.at[pl.ds(NS * RPT, REXTRA)])

    plsc.subcore_barrier()

    ebase = s * EPT

    def start(srcb, dstb, rowsb, semb, e0):
        pltpu.sync_copy(src_hbm.at[pl.ds(e0, CH)], srcb)
        pltpu.sync_copy(dst_hbm.at[pl.ds(e0, CH)], dstb)
        for i in range(CH // 16):
            sl = pl.ds(i * 16, 16)
            srcb[sl] = srcb[sl] + roff
        pltpu.async_copy(x_hbm.at[srcb], rowsb, semb)

    def finish(srcb, dstb, rowsb, semb):
        pltpu.make_async_copy(x_hbm.at[srcb], rowsb, semb).wait()
        pltpu.sync_copy(rowsb, acc.at[dstb], add=True)

    start(src0, dst0, rows0, sem0, ebase)
    start(src1, dst1, rows1, sem1, ebase + CH)

    @pl.loop(0, NFULL // 3)
    def _chunks(t):
        e0 = ebase + t * (3 * CH)
        pltpu.make_async_copy(x_hbm.at[src0], rows0, sem0).wait()
        start(src2, dst2, rows2, sem2, e0 + 2 * CH)
        pltpu.sync_copy(rows0, acc.at[dst0], add=True)

        pltpu.make_async_copy(x_hbm.at[src1], rows1, sem1).wait()
        start(src0, dst0, rows0, sem0, e0 + 3 * CH)
        pltpu.sync_copy(rows1, acc.at[dst1], add=True)

        pltpu.make_async_copy(x_hbm.at[src2], rows2, sem2).wait()
        start(src1, dst1, rows1, sem1, e0 + 4 * CH)
        pltpu.sync_copy(rows2, acc.at[dst2], add=True)

    finish(src0, dst0, rows0, sem0)
    finish(src1, dst1, rows1, sem1)

    if TAIL:
        e0 = ebase + NFULL * CH
        pltpu.sync_copy(src_hbm.at[pl.ds(e0, TAIL)], tsrc)
        pltpu.sync_copy(dst_hbm.at[pl.ds(e0, TAIL)], tdst)
        for i in range(TAIL // 16):
            sl = pl.ds(i * 16, 16)
            tsrc[sl] = tsrc[sl] + roff
        pltpu.async_copy(x_hbm.at[tsrc], trows, sem0).wait()
        pltpu.sync_copy(trows, acc.at[tdst], add=True)

    plsc.subcore_barrier()
    pltpu.sync_copy(acc.at[pl.ds(r0, RPT)], out_hbm.at[pl.ds(roff + r0, RPT)])

    @pl.when(s == NS - 1)
    def _out_extra():
        pltpu.sync_copy(acc.at[pl.ds(NS * RPT, REXTRA)],
                        out_hbm.at[pl.ds(roff + NS * RPT, REXTRA)])


@jax.jit
def _sc_agg(x_cat, src, dst):
    mesh = plsc.VectorSubcoreMesh(core_axis_name="c", subcore_axis_name="s")
    return pl.kernel(
        _sc_agg_body,
        out_type=jax.ShapeDtypeStruct((2 * N, H), jnp.float32),
        mesh=mesh,
        scratch_types=[
            pltpu.VMEM((CH,), jnp.int32),
            pltpu.VMEM((CH,), jnp.int32),
            pltpu.VMEM((CH, H), jnp.float32),
            pltpu.VMEM((CH,), jnp.int32),
            pltpu.VMEM((CH,), jnp.int32),
            pltpu.VMEM((CH, H), jnp.float32),
            pltpu.VMEM((CH,), jnp.int32),
            pltpu.VMEM((CH,), jnp.int32),
            pltpu.VMEM((CH, H), jnp.float32),
            pltpu.VMEM((max(TAIL, 16),), jnp.int32),
            pltpu.VMEM((max(TAIL, 16),), jnp.int32),
            pltpu.VMEM((max(TAIL, 16), H), jnp.float32),
            pltpu.VMEM_SHARED((N, H), jnp.float32),
            pltpu.SemaphoreType.DMA,
            pltpu.SemaphoreType.DMA,
            pltpu.SemaphoreType.DMA,
        ],
    )(x_cat, src, dst)


def _dense_body_split(hin_ref, w_ref, b_ref, out_ref):
    hl = hin_ref[0]
    hh = hin_ref[1]
    h = (jnp.dot(hl, w_ref[:H, :], preferred_element_type=jnp.float32)
         + jnp.dot(hh, w_ref[H:, :], preferred_element_type=jnp.float32)
         + b_ref[...])
    o = h * jax.nn.sigmoid(h)
    out_ref[0] = o[:, :H]
    out_ref[1] = o[:, H:]


def _dense_body_last(hin_ref, w_ref, b_ref, out_ref):
    hl = hin_ref[0]
    hh = hin_ref[1]
    h = (jnp.dot(hl, w_ref[:H, :], preferred_element_type=jnp.float32)
         + jnp.dot(hh, w_ref[H:, :], preferred_element_type=jnp.float32)
         + b_ref[...])
    out_ref[...] = h * jax.nn.sigmoid(h)


_RB = 2000


@functools.partial(jax.jit, static_argnames=("last",))
def _dense(hin2, w, b2, last=False):
    grid = (N // _RB,)
    in_specs = [
        pl.BlockSpec((2, _RB, H), lambda i: (0, i, 0)),
        pl.BlockSpec((D, D), lambda i: (0, 0)),
        pl.BlockSpec((1, D), lambda i: (0, 0)),
    ]
    if last:
        return pl.pallas_call(
            _dense_body_last,
            grid=grid,
            in_specs=in_specs,
            out_specs=pl.BlockSpec((_RB, D), lambda i: (i, 0)),
            out_shape=jax.ShapeDtypeStruct((N, D), jnp.float32),
        )(hin2, w, b2)
    return pl.pallas_call(
        _dense_body_split,
        grid=grid,
        in_specs=in_specs,
        out_specs=pl.BlockSpec((2, _RB, H), lambda i: (0, i, 0)),
        out_shape=jax.ShapeDtypeStruct((2, N, H), jnp.float32),
    )(hin2, w, b2)


def kernel(x, edge_index, W0, b0, W1, b1, W2, b2):
    edges = edge_index.astype(jnp.int32)
    src = edges[0]
    dst = edges[1]
    h2 = x.reshape(N, 2, H).transpose(1, 0, 2)
    params = [(W0, b0), (W1, b1), (W2, b2)]
    for li, (w, b) in enumerate(params):
        hin = _sc_agg(h2.reshape(2 * N, H), src, dst)
        h2 = _dense(hin.reshape(2, N, H), w, b.reshape(1, D), last=(li == 2))
    return h2
